# baseline (speedup 1.0000x reference)
import functools

import jax
import jax.numpy as jnp
from jax.experimental import pallas as pl

_BE = 512
_BN = 512
_PREC = jax.lax.Precision.HIGHEST


def _proj_kernel(h_ref, wt_ref, asrc_ref, adst_ref, zxp_ref, zal_ref, zar_ref):
    h = h_ref[...]
    zxp_ref[...] = jnp.dot(h, wt_ref[...], preferred_element_type=jnp.float32, precision=_PREC)
    zal_ref[...] = jnp.dot(h, asrc_ref[...], preferred_element_type=jnp.float32, precision=_PREC)
    zar_ref[...] = jnp.dot(h, adst_ref[...], preferred_element_type=jnp.float32, precision=_PREC)


def _norm_proj_kernel(num_ref, den_ref, rep_ref, b_ref, wt_ref, asrc_ref, adst_ref,
                      zxp_ref, zal_ref, zar_ref):
    den = jnp.dot(den_ref[...], rep_ref[...], preferred_element_type=jnp.float32, precision=_PREC)
    h = num_ref[...] / (den + 1e-16) + b_ref[...]
    h = jnp.where(h > 0, h, jnp.exp(h) - 1.0)
    zxp_ref[...] = jnp.dot(h, wt_ref[...], preferred_element_type=jnp.float32, precision=_PREC)
    zal_ref[...] = jnp.dot(h, asrc_ref[...], preferred_element_type=jnp.float32, precision=_PREC)
    zar_ref[...] = jnp.dot(h, adst_ref[...], preferred_element_type=jnp.float32, precision=_PREC)


def _edge_kernel(zxp_ref, zal_ref, zar_ref, src_ref, dst_ref, rep_ref,
                 msg_ref, w_ref, *, n_chunks, chunk):
    src = src_ref[...]
    dst = dst_ref[...]
    b = src.shape[0]
    f = zxp_ref.shape[1]
    hh = zal_ref.shape[1]

    def body(k, carry):
        xp, al, ar = carry
        base = k * chunk
        col = jax.lax.broadcasted_iota(jnp.int32, (b, chunk), 1) + base
        ohs = (src == col).astype(jnp.float32)
        ohd = (dst == col).astype(jnp.float32)
        xp = xp + jnp.dot(ohs, zxp_ref[pl.ds(base, chunk), :],
                          preferred_element_type=jnp.float32, precision=_PREC)
        al = al + jnp.dot(ohs, zal_ref[pl.ds(base, chunk), :],
                          preferred_element_type=jnp.float32, precision=_PREC)
        ar = ar + jnp.dot(ohd, zar_ref[pl.ds(base, chunk), :],
                          preferred_element_type=jnp.float32, precision=_PREC)
        return xp, al, ar

    init = (jnp.zeros((b, f), jnp.float32),
            jnp.zeros((b, hh), jnp.float32),
            jnp.zeros((b, hh), jnp.float32))
    xp, al, ar = jax.lax.fori_loop(0, n_chunks, body, init)
    a = al + ar
    a = jnp.where(a > 0, a, 0.2 * a)
    w = jnp.exp(a)
    msg_ref[...] = jnp.dot(w, rep_ref[...], preferred_element_type=jnp.float32,
                           precision=_PREC) * xp
    w_ref[...] = w


def _scatter_kernel(msg_ref, w_ref, dst_ref, num_ref, den_ref, *, bn):
    e = pl.program_id(1)

    @pl.when(e == 0)
    def _():
        num_ref[...] = jnp.zeros_like(num_ref)
        den_ref[...] = jnp.zeros_like(den_ref)

    n = pl.program_id(0)
    dst = dst_ref[...]
    b = dst.shape[0]
    row = jax.lax.broadcasted_iota(jnp.int32, (b, bn), 1) + n * bn
    oh = (dst == row).astype(jnp.float32)
    dn = (((0,), (0,)), ((), ()))
    num_ref[...] += jax.lax.dot_general(oh, msg_ref[...], dn,
                                        preferred_element_type=jnp.float32, precision=_PREC)
    den_ref[...] += jax.lax.dot_general(oh, w_ref[...], dn,
                                        preferred_element_type=jnp.float32, precision=_PREC)


def _final_kernel(num_ref, den_ref, rep_ref, b_ref, out_ref):
    den = jnp.dot(den_ref[...], rep_ref[...], preferred_element_type=jnp.float32, precision=_PREC)
    logits = num_ref[...] / (den + 1e-16) + b_ref[...]
    m = jnp.max(logits, axis=-1, keepdims=True)
    ex = jnp.exp(logits - m)
    out_ref[...] = ex / jnp.sum(ex, axis=-1, keepdims=True)


def _edge_scatter(zxp, zal, zar, src2d, dst2d, rep, n_pad, e_pad):
    f = zxp.shape[1]
    hh = zal.shape[1]
    eb = e_pad // _BE
    nt = n_pad // _BN
    msg, w = pl.pallas_call(
        functools.partial(_edge_kernel, n_chunks=n_pad // _BN, chunk=_BN),
        grid=(eb,),
        in_specs=[
            pl.BlockSpec((n_pad, f), lambda i: (0, 0)),
            pl.BlockSpec((n_pad, hh), lambda i: (0, 0)),
            pl.BlockSpec((n_pad, hh), lambda i: (0, 0)),
            pl.BlockSpec((_BE, 1), lambda i: (i, 0)),
            pl.BlockSpec((_BE, 1), lambda i: (i, 0)),
            pl.BlockSpec((hh, f), lambda i: (0, 0)),
        ],
        out_specs=[
            pl.BlockSpec((_BE, f), lambda i: (i, 0)),
            pl.BlockSpec((_BE, hh), lambda i: (i, 0)),
        ],
        out_shape=[
            jax.ShapeDtypeStruct((e_pad, f), jnp.float32),
            jax.ShapeDtypeStruct((e_pad, hh), jnp.float32),
        ],
    )(zxp, zal, zar, src2d, dst2d, rep)

    num, den = pl.pallas_call(
        functools.partial(_scatter_kernel, bn=_BN),
        grid=(nt, eb),
        in_specs=[
            pl.BlockSpec((_BE, f), lambda n, e: (e, 0)),
            pl.BlockSpec((_BE, hh), lambda n, e: (e, 0)),
            pl.BlockSpec((_BE, 1), lambda n, e: (e, 0)),
        ],
        out_specs=[
            pl.BlockSpec((_BN, f), lambda n, e: (n, 0)),
            pl.BlockSpec((_BN, hh), lambda n, e: (n, 0)),
        ],
        out_shape=[
            jax.ShapeDtypeStruct((n_pad, f), jnp.float32),
            jax.ShapeDtypeStruct((n_pad, hh), jnp.float32),
        ],
    )(msg, w, dst2d)
    return num, den


def _attn_mats(a_src, a_dst, h, c, h_pad):
    eye = jnp.eye(h, h_pad, dtype=jnp.float32)
    ms = jnp.einsum('hc,hg->hcg', a_src[0], eye).reshape(h * c, h_pad)
    md = jnp.einsum('hc,hg->hcg', a_dst[0], eye).reshape(h * c, h_pad)
    return ms, md


@jax.jit
def kernel(x, edge_index, W1, a_src1, a_dst1, b1, W2, a_src2, a_dst2, b2):
    n, d_in = x.shape
    e = edge_index.shape[1]
    h1, c1 = a_src1.shape[1], a_src1.shape[2]
    h2, c2 = a_src2.shape[1], a_src2.shape[2]
    f1, f2 = h1 * c1, h2 * c2
    hp = 8

    n_pad = ((n + _BN - 1) // _BN) * _BN
    nt = n_pad // _BN

    src0 = edge_index[0]
    dst0 = jnp.where(src0 != edge_index[1], edge_index[1], jnp.int32(n))
    loops = jnp.arange(n, dtype=jnp.int32)
    src = jnp.concatenate([src0, loops])
    dst = jnp.concatenate([dst0, loops])
    e_tot = src.shape[0]
    e_pad = ((e_tot + _BE - 1) // _BE) * _BE
    pad = e_pad - e_tot
    src = jnp.concatenate([src, jnp.zeros((pad,), jnp.int32)])
    dst = jnp.concatenate([dst, jnp.full((pad,), n_pad - 1, jnp.int32)])
    src2d = src.reshape(e_pad, 1)
    dst2d = dst.reshape(e_pad, 1)

    x_pad = jnp.pad(x, ((0, n_pad - n), (0, 0)))

    ms1, md1 = _attn_mats(a_src1, a_dst1, h1, c1, hp)
    ms2, md2 = _attn_mats(a_src2, a_dst2, h2, c2, hp)
    wt1 = W1.T
    asrc1 = wt1 @ ms1
    adst1 = wt1 @ md1
    wt2 = W2.T
    asrc2 = wt2 @ ms2
    adst2 = wt2 @ md2
    rep1 = jnp.repeat(jnp.eye(hp, dtype=jnp.float32), c1, axis=1)[:, :f1]
    rep2 = jnp.concatenate(
        [jnp.ones((h2, f2), jnp.float32), jnp.zeros((hp - h2, f2), jnp.float32)], axis=0)
    b1r = b1.reshape(1, f1)
    b2r = b2.reshape(1, f2)

    zxp1, zal1, zar1 = pl.pallas_call(
        _proj_kernel,
        grid=(nt,),
        in_specs=[
            pl.BlockSpec((_BN, d_in), lambda i: (i, 0)),
            pl.BlockSpec((d_in, f1), lambda i: (0, 0)),
            pl.BlockSpec((d_in, hp), lambda i: (0, 0)),
            pl.BlockSpec((d_in, hp), lambda i: (0, 0)),
        ],
        out_specs=[
            pl.BlockSpec((_BN, f1), lambda i: (i, 0)),
            pl.BlockSpec((_BN, hp), lambda i: (i, 0)),
            pl.BlockSpec((_BN, hp), lambda i: (i, 0)),
        ],
        out_shape=[
            jax.ShapeDtypeStruct((n_pad, f1), jnp.float32),
            jax.ShapeDtypeStruct((n_pad, hp), jnp.float32),
            jax.ShapeDtypeStruct((n_pad, hp), jnp.float32),
        ],
    )(x_pad, wt1, asrc1, adst1)

    num1, den1 = _edge_scatter(zxp1, zal1, zar1, src2d, dst2d, rep1, n_pad, e_pad)

    zxp2, zal2, zar2 = pl.pallas_call(
        _norm_proj_kernel,
        grid=(nt,),
        in_specs=[
            pl.BlockSpec((_BN, f1), lambda i: (i, 0)),
            pl.BlockSpec((_BN, hp), lambda i: (i, 0)),
            pl.BlockSpec((hp, f1), lambda i: (0, 0)),
            pl.BlockSpec((1, f1), lambda i: (0, 0)),
            pl.BlockSpec((f1, f2), lambda i: (0, 0)),
            pl.BlockSpec((f1, hp), lambda i: (0, 0)),
            pl.BlockSpec((f1, hp), lambda i: (0, 0)),
        ],
        out_specs=[
            pl.BlockSpec((_BN, f2), lambda i: (i, 0)),
            pl.BlockSpec((_BN, hp), lambda i: (i, 0)),
            pl.BlockSpec((_BN, hp), lambda i: (i, 0)),
        ],
        out_shape=[
            jax.ShapeDtypeStruct((n_pad, f2), jnp.float32),
            jax.ShapeDtypeStruct((n_pad, hp), jnp.float32),
            jax.ShapeDtypeStruct((n_pad, hp), jnp.float32),
        ],
    )(num1, den1, rep1, b1r, wt2, asrc2, adst2)

    num2, den2 = _edge_scatter(zxp2, zal2, zar2, src2d, dst2d, rep2, n_pad, e_pad)

    out = pl.pallas_call(
        _final_kernel,
        grid=(nt,),
        in_specs=[
            pl.BlockSpec((_BN, f2), lambda i: (i, 0)),
            pl.BlockSpec((_BN, hp), lambda i: (i, 0)),
            pl.BlockSpec((hp, f2), lambda i: (0, 0)),
            pl.BlockSpec((1, f2), lambda i: (0, 0)),
        ],
        out_specs=pl.BlockSpec((_BN, f2), lambda i: (i, 0)),
        out_shape=jax.ShapeDtypeStruct((n_pad, f2), jnp.float32),
    )(num2, den2, rep2, b2r)

    return out[:n]


# speedup vs baseline: 2.9947x; 2.9947x over previous
import functools

import jax
import jax.numpy as jnp
from jax.experimental import pallas as pl

_BE = 512
_BN = 512
_PREC = jax.lax.Precision.HIGHEST
_PREC_OH = jax.lax.Precision.DEFAULT


def _proj_kernel(h_ref, wt_ref, asrc_ref, adst_ref, zxp_ref, zal_ref, zar_ref):
    h = h_ref[...]
    zxp_ref[...] = jnp.dot(h, wt_ref[...], preferred_element_type=jnp.float32, precision=_PREC)
    zal_ref[...] = jnp.dot(h, asrc_ref[...], preferred_element_type=jnp.float32, precision=_PREC)
    zar_ref[...] = jnp.dot(h, adst_ref[...], preferred_element_type=jnp.float32, precision=_PREC)


def _norm_proj_kernel(num_ref, den_ref, rep_ref, b_ref, wt_ref, asrc_ref, adst_ref,
                      zxp_ref, zal_ref, zar_ref):
    den = jnp.dot(den_ref[...], rep_ref[...], preferred_element_type=jnp.float32, precision=_PREC)
    h = num_ref[...] / (den + 1e-16) + b_ref[...]
    h = jnp.where(h > 0, h, jnp.exp(h) - 1.0)
    zxp_ref[...] = jnp.dot(h, wt_ref[...], preferred_element_type=jnp.float32, precision=_PREC)
    zal_ref[...] = jnp.dot(h, asrc_ref[...], preferred_element_type=jnp.float32, precision=_PREC)
    zar_ref[...] = jnp.dot(h, adst_ref[...], preferred_element_type=jnp.float32, precision=_PREC)


def _edge_kernel(zxp_ref, zal_ref, zar_ref, src_ref, dst_ref, rep_ref,
                 msg_ref, w_ref, *, n_chunks, chunk):
    src = src_ref[...]
    dst = dst_ref[...]
    b = src.shape[0]
    f = zxp_ref.shape[1]
    hh = zal_ref.shape[1]

    def body(k, carry):
        xp, al, ar = carry
        base = k * chunk
        col = jax.lax.broadcasted_iota(jnp.int32, (b, chunk), 1) + base
        ohs = (src == col).astype(jnp.float32)
        ohd = (dst == col).astype(jnp.float32)
        xp = xp + jnp.dot(ohs, zxp_ref[pl.ds(base, chunk), :],
                          preferred_element_type=jnp.float32, precision=_PREC_OH)
        al = al + jnp.dot(ohs, zal_ref[pl.ds(base, chunk), :],
                          preferred_element_type=jnp.float32, precision=_PREC_OH)
        ar = ar + jnp.dot(ohd, zar_ref[pl.ds(base, chunk), :],
                          preferred_element_type=jnp.float32, precision=_PREC_OH)
        return xp, al, ar

    init = (jnp.zeros((b, f), jnp.float32),
            jnp.zeros((b, hh), jnp.float32),
            jnp.zeros((b, hh), jnp.float32))
    xp, al, ar = jax.lax.fori_loop(0, n_chunks, body, init)
    a = al + ar
    a = jnp.where(a > 0, a, 0.2 * a)
    w = jnp.exp(a)
    msg_ref[...] = jnp.dot(w, rep_ref[...], preferred_element_type=jnp.float32,
                           precision=_PREC) * xp
    w_ref[...] = w


def _scatter_kernel(msg_ref, w_ref, dst_ref, num_ref, den_ref, *, bn):
    e = pl.program_id(1)

    @pl.when(e == 0)
    def _():
        num_ref[...] = jnp.zeros_like(num_ref)
        den_ref[...] = jnp.zeros_like(den_ref)

    n = pl.program_id(0)
    dst = dst_ref[...]
    b = dst.shape[0]
    row = jax.lax.broadcasted_iota(jnp.int32, (b, bn), 1) + n * bn
    oh = (dst == row).astype(jnp.float32)
    dn = (((0,), (0,)), ((), ()))
    num_ref[...] += jax.lax.dot_general(oh, msg_ref[...], dn,
                                        preferred_element_type=jnp.float32, precision=_PREC_OH)
    den_ref[...] += jax.lax.dot_general(oh, w_ref[...], dn,
                                        preferred_element_type=jnp.float32, precision=_PREC_OH)


def _final_kernel(num_ref, den_ref, rep_ref, b_ref, out_ref):
    den = jnp.dot(den_ref[...], rep_ref[...], preferred_element_type=jnp.float32, precision=_PREC)
    logits = num_ref[...] / (den + 1e-16) + b_ref[...]
    m = jnp.max(logits, axis=-1, keepdims=True)
    ex = jnp.exp(logits - m)
    out_ref[...] = ex / jnp.sum(ex, axis=-1, keepdims=True)


def _edge_scatter(zxp, zal, zar, src2d, dst2d, rep, n_pad, e_pad):
    f = zxp.shape[1]
    hh = zal.shape[1]
    eb = e_pad // _BE
    nt = n_pad // _BN
    msg, w = pl.pallas_call(
        functools.partial(_edge_kernel, n_chunks=n_pad // _BN, chunk=_BN),
        grid=(eb,),
        in_specs=[
            pl.BlockSpec((n_pad, f), lambda i: (0, 0)),
            pl.BlockSpec((n_pad, hh), lambda i: (0, 0)),
            pl.BlockSpec((n_pad, hh), lambda i: (0, 0)),
            pl.BlockSpec((_BE, 1), lambda i: (i, 0)),
            pl.BlockSpec((_BE, 1), lambda i: (i, 0)),
            pl.BlockSpec((hh, f), lambda i: (0, 0)),
        ],
        out_specs=[
            pl.BlockSpec((_BE, f), lambda i: (i, 0)),
            pl.BlockSpec((_BE, hh), lambda i: (i, 0)),
        ],
        out_shape=[
            jax.ShapeDtypeStruct((e_pad, f), jnp.float32),
            jax.ShapeDtypeStruct((e_pad, hh), jnp.float32),
        ],
    )(zxp, zal, zar, src2d, dst2d, rep)

    num, den = pl.pallas_call(
        functools.partial(_scatter_kernel, bn=_BN),
        grid=(nt, eb),
        in_specs=[
            pl.BlockSpec((_BE, f), lambda n, e: (e, 0)),
            pl.BlockSpec((_BE, hh), lambda n, e: (e, 0)),
            pl.BlockSpec((_BE, 1), lambda n, e: (e, 0)),
        ],
        out_specs=[
            pl.BlockSpec((_BN, f), lambda n, e: (n, 0)),
            pl.BlockSpec((_BN, hh), lambda n, e: (n, 0)),
        ],
        out_shape=[
            jax.ShapeDtypeStruct((n_pad, f), jnp.float32),
            jax.ShapeDtypeStruct((n_pad, hh), jnp.float32),
        ],
    )(msg, w, dst2d)
    return num, den


def _attn_mats(a_src, a_dst, h, c, h_pad):
    eye = jnp.eye(h, h_pad, dtype=jnp.float32)
    ms = jnp.einsum('hc,hg->hcg', a_src[0], eye).reshape(h * c, h_pad)
    md = jnp.einsum('hc,hg->hcg', a_dst[0], eye).reshape(h * c, h_pad)
    return ms, md


@jax.jit
def kernel(x, edge_index, W1, a_src1, a_dst1, b1, W2, a_src2, a_dst2, b2):
    n, d_in = x.shape
    e = edge_index.shape[1]
    h1, c1 = a_src1.shape[1], a_src1.shape[2]
    h2, c2 = a_src2.shape[1], a_src2.shape[2]
    f1, f2 = h1 * c1, h2 * c2
    hp = 8

    n_pad = ((n + _BN - 1) // _BN) * _BN
    nt = n_pad // _BN

    src0 = edge_index[0]
    dst0 = jnp.where(src0 != edge_index[1], edge_index[1], jnp.int32(n))
    loops = jnp.arange(n, dtype=jnp.int32)
    src = jnp.concatenate([src0, loops])
    dst = jnp.concatenate([dst0, loops])
    e_tot = src.shape[0]
    e_pad = ((e_tot + _BE - 1) // _BE) * _BE
    pad = e_pad - e_tot
    src = jnp.concatenate([src, jnp.zeros((pad,), jnp.int32)])
    dst = jnp.concatenate([dst, jnp.full((pad,), n_pad - 1, jnp.int32)])
    src2d = src.reshape(e_pad, 1)
    dst2d = dst.reshape(e_pad, 1)

    x_pad = jnp.pad(x, ((0, n_pad - n), (0, 0)))

    ms1, md1 = _attn_mats(a_src1, a_dst1, h1, c1, hp)
    ms2, md2 = _attn_mats(a_src2, a_dst2, h2, c2, hp)
    wt1 = W1.T
    asrc1 = wt1 @ ms1
    adst1 = wt1 @ md1
    wt2 = W2.T
    asrc2 = wt2 @ ms2
    adst2 = wt2 @ md2
    rep1 = jnp.repeat(jnp.eye(hp, dtype=jnp.float32), c1, axis=1)[:, :f1]
    rep2 = jnp.concatenate(
        [jnp.ones((h2, f2), jnp.float32), jnp.zeros((hp - h2, f2), jnp.float32)], axis=0)
    b1r = b1.reshape(1, f1)
    b2r = b2.reshape(1, f2)

    zxp1, zal1, zar1 = pl.pallas_call(
        _proj_kernel,
        grid=(nt,),
        in_specs=[
            pl.BlockSpec((_BN, d_in), lambda i: (i, 0)),
            pl.BlockSpec((d_in, f1), lambda i: (0, 0)),
            pl.BlockSpec((d_in, hp), lambda i: (0, 0)),
            pl.BlockSpec((d_in, hp), lambda i: (0, 0)),
        ],
        out_specs=[
            pl.BlockSpec((_BN, f1), lambda i: (i, 0)),
            pl.BlockSpec((_BN, hp), lambda i: (i, 0)),
            pl.BlockSpec((_BN, hp), lambda i: (i, 0)),
        ],
        out_shape=[
            jax.ShapeDtypeStruct((n_pad, f1), jnp.float32),
            jax.ShapeDtypeStruct((n_pad, hp), jnp.float32),
            jax.ShapeDtypeStruct((n_pad, hp), jnp.float32),
        ],
    )(x_pad, wt1, asrc1, adst1)

    num1, den1 = _edge_scatter(zxp1, zal1, zar1, src2d, dst2d, rep1, n_pad, e_pad)

    zxp2, zal2, zar2 = pl.pallas_call(
        _norm_proj_kernel,
        grid=(nt,),
        in_specs=[
            pl.BlockSpec((_BN, f1), lambda i: (i, 0)),
            pl.BlockSpec((_BN, hp), lambda i: (i, 0)),
            pl.BlockSpec((hp, f1), lambda i: (0, 0)),
            pl.BlockSpec((1, f1), lambda i: (0, 0)),
            pl.BlockSpec((f1, f2), lambda i: (0, 0)),
            pl.BlockSpec((f1, hp), lambda i: (0, 0)),
            pl.BlockSpec((f1, hp), lambda i: (0, 0)),
        ],
        out_specs=[
            pl.BlockSpec((_BN, f2), lambda i: (i, 0)),
            pl.BlockSpec((_BN, hp), lambda i: (i, 0)),
            pl.BlockSpec((_BN, hp), lambda i: (i, 0)),
        ],
        out_shape=[
            jax.ShapeDtypeStruct((n_pad, f2), jnp.float32),
            jax.ShapeDtypeStruct((n_pad, hp), jnp.float32),
            jax.ShapeDtypeStruct((n_pad, hp), jnp.float32),
        ],
    )(num1, den1, rep1, b1r, wt2, asrc2, adst2)

    num2, den2 = _edge_scatter(zxp2, zal2, zar2, src2d, dst2d, rep2, n_pad, e_pad)

    out = pl.pallas_call(
        _final_kernel,
        grid=(nt,),
        in_specs=[
            pl.BlockSpec((_BN, f2), lambda i: (i, 0)),
            pl.BlockSpec((_BN, hp), lambda i: (i, 0)),
            pl.BlockSpec((hp, f2), lambda i: (0, 0)),
            pl.BlockSpec((1, f2), lambda i: (0, 0)),
        ],
        out_specs=pl.BlockSpec((_BN, f2), lambda i: (i, 0)),
        out_shape=jax.ShapeDtypeStruct((n_pad, f2), jnp.float32),
    )(num2, den2, rep2, b2r)

    return out[:n]


# speedup vs baseline: 4.8728x; 1.6271x over previous
import functools

import jax
import jax.numpy as jnp
from jax.experimental import pallas as pl

_BE = 512
_BN = 512
_PREC = jax.lax.Precision.HIGHEST
_PREC_OH = jax.lax.Precision.DEFAULT


def _proj_kernel(h_ref, wsrc_ref, adst_ref, zsrc_ref, zar_ref):
    h = h_ref[...]
    zsrc_ref[...] = jnp.dot(h, wsrc_ref[...], preferred_element_type=jnp.float32, precision=_PREC)
    zar_ref[...] = jnp.dot(h, adst_ref[...], preferred_element_type=jnp.float32, precision=_PREC)


def _norm_proj_kernel(num_ref, den_ref, rep_ref, b_ref, wsrc_ref, adst_ref,
                      zsrc_ref, zar_ref):
    den = jnp.dot(den_ref[...], rep_ref[...], preferred_element_type=jnp.float32, precision=_PREC)
    h = num_ref[...] / (den + 1e-16) + b_ref[...]
    h = jnp.where(h > 0, h, jnp.exp(h) - 1.0)
    zsrc_ref[...] = jnp.dot(h, wsrc_ref[...], preferred_element_type=jnp.float32, precision=_PREC)
    zar_ref[...] = jnp.dot(h, adst_ref[...], preferred_element_type=jnp.float32, precision=_PREC)


def _edge_scatter_kernel(zsrc_ref, zar_ref, src_ref, dst_ref, rep_ref, acc_ref,
                         *, n_chunks, chunk, f):
    e = pl.program_id(0)

    @pl.when(e == 0)
    def _():
        acc_ref[...] = jnp.zeros_like(acc_ref)

    src = src_ref[...]
    dst = dst_ref[...]
    b = src.shape[0]
    fh = zsrc_ref.shape[1]
    hp = zar_ref.shape[1]

    def gbody(k, carry):
        gs, ar = carry
        base = k * chunk
        col = jax.lax.broadcasted_iota(jnp.int32, (b, chunk), 1) + base
        ohs = (src == col).astype(jnp.float32)
        ohd = (dst == col).astype(jnp.float32)
        gs = gs + jnp.dot(ohs, zsrc_ref[pl.ds(base, chunk), :],
                          preferred_element_type=jnp.float32, precision=_PREC_OH)
        ar = ar + jnp.dot(ohd, zar_ref[pl.ds(base, chunk), :],
                          preferred_element_type=jnp.float32, precision=_PREC_OH)
        return gs, ar

    init = (jnp.zeros((b, fh), jnp.float32), jnp.zeros((b, hp), jnp.float32))
    gs, ar = jax.lax.fori_loop(0, n_chunks, gbody, init)
    xp = gs[:, :f]
    a = gs[:, f:] + ar
    a = jnp.where(a > 0, a, 0.2 * a)
    w = jnp.exp(a)
    mw = jnp.concatenate(
        [jnp.dot(w, rep_ref[...], preferred_element_type=jnp.float32,
                 precision=_PREC_OH) * xp, w], axis=1)
    dn = (((0,), (0,)), ((), ()))

    def sbody(k, _):
        base = k * chunk
        col = jax.lax.broadcasted_iota(jnp.int32, (b, chunk), 1) + base
        ohd = (dst == col).astype(jnp.float32)
        acc_ref[pl.ds(base, chunk), :] += jax.lax.dot_general(
            ohd, mw, dn, preferred_element_type=jnp.float32, precision=_PREC_OH)
        return 0

    jax.lax.fori_loop(0, n_chunks, sbody, 0)


def _final_kernel(num_ref, den_ref, rep_ref, b_ref, out_ref):
    den = jnp.dot(den_ref[...], rep_ref[...], preferred_element_type=jnp.float32, precision=_PREC)
    logits = num_ref[...] / (den + 1e-16) + b_ref[...]
    m = jnp.max(logits, axis=-1, keepdims=True)
    ex = jnp.exp(logits - m)
    out_ref[...] = ex / jnp.sum(ex, axis=-1, keepdims=True)


def _edge_scatter(zsrc, zar, src2d, dst2d, rep, n_pad, e_pad, f, hp):
    eb = e_pad // _BE
    acc = pl.pallas_call(
        functools.partial(_edge_scatter_kernel, n_chunks=n_pad // _BN, chunk=_BN, f=f),
        grid=(eb,),
        in_specs=[
            pl.BlockSpec((n_pad, f + hp), lambda i: (0, 0)),
            pl.BlockSpec((n_pad, hp), lambda i: (0, 0)),
            pl.BlockSpec((_BE, 1), lambda i: (i, 0)),
            pl.BlockSpec((_BE, 1), lambda i: (i, 0)),
            pl.BlockSpec((hp, f), lambda i: (0, 0)),
        ],
        out_specs=pl.BlockSpec((n_pad, f + hp), lambda i: (0, 0)),
        out_shape=jax.ShapeDtypeStruct((n_pad, f + hp), jnp.float32),
    )(zsrc, zar, src2d, dst2d, rep)
    return acc[:, :f], acc[:, f:]


def _attn_mats(a_src, a_dst, h, c, h_pad):
    eye = jnp.eye(h, h_pad, dtype=jnp.float32)
    ms = jnp.einsum('hc,hg->hcg', a_src[0], eye).reshape(h * c, h_pad)
    md = jnp.einsum('hc,hg->hcg', a_dst[0], eye).reshape(h * c, h_pad)
    return ms, md


@jax.jit
def kernel(x, edge_index, W1, a_src1, a_dst1, b1, W2, a_src2, a_dst2, b2):
    n, d_in = x.shape
    h1, c1 = a_src1.shape[1], a_src1.shape[2]
    h2, c2 = a_src2.shape[1], a_src2.shape[2]
    f1, f2 = h1 * c1, h2 * c2
    hp = 8

    n_pad = ((n + _BN - 1) // _BN) * _BN
    nt = n_pad // _BN

    src0 = edge_index[0]
    dst0 = jnp.where(src0 != edge_index[1], edge_index[1], jnp.int32(n))
    loops = jnp.arange(n, dtype=jnp.int32)
    src = jnp.concatenate([src0, loops])
    dst = jnp.concatenate([dst0, loops])
    e_tot = src.shape[0]
    e_pad = ((e_tot + _BE - 1) // _BE) * _BE
    pad = e_pad - e_tot
    src = jnp.concatenate([src, jnp.zeros((pad,), jnp.int32)])
    dst = jnp.concatenate([dst, jnp.full((pad,), n_pad - 1, jnp.int32)])
    src2d = src.reshape(e_pad, 1)
    dst2d = dst.reshape(e_pad, 1)

    x_pad = jnp.pad(x, ((0, n_pad - n), (0, 0)))

    ms1, md1 = _attn_mats(a_src1, a_dst1, h1, c1, hp)
    ms2, md2 = _attn_mats(a_src2, a_dst2, h2, c2, hp)
    wt1 = W1.T
    wsrc1 = jnp.concatenate([wt1, wt1 @ ms1], axis=1)
    adst1 = wt1 @ md1
    wt2 = W2.T
    wsrc2 = jnp.concatenate([wt2, wt2 @ ms2], axis=1)
    adst2 = wt2 @ md2
    rep1 = jnp.repeat(jnp.eye(hp, dtype=jnp.float32), c1, axis=1)[:, :f1]
    rep2 = jnp.concatenate(
        [jnp.ones((h2, f2), jnp.float32), jnp.zeros((hp - h2, f2), jnp.float32)], axis=0)
    b1r = b1.reshape(1, f1)
    b2r = b2.reshape(1, f2)

    zsrc1, zar1 = pl.pallas_call(
        _proj_kernel,
        grid=(nt,),
        in_specs=[
            pl.BlockSpec((_BN, d_in), lambda i: (i, 0)),
            pl.BlockSpec((d_in, f1 + hp), lambda i: (0, 0)),
            pl.BlockSpec((d_in, hp), lambda i: (0, 0)),
        ],
        out_specs=[
            pl.BlockSpec((_BN, f1 + hp), lambda i: (i, 0)),
            pl.BlockSpec((_BN, hp), lambda i: (i, 0)),
        ],
        out_shape=[
            jax.ShapeDtypeStruct((n_pad, f1 + hp), jnp.float32),
            jax.ShapeDtypeStruct((n_pad, hp), jnp.float32),
        ],
    )(x_pad, wsrc1, adst1)

    num1, den1 = _edge_scatter(zsrc1, zar1, src2d, dst2d, rep1, n_pad, e_pad, f1, hp)

    zsrc2, zar2 = pl.pallas_call(
        _norm_proj_kernel,
        grid=(nt,),
        in_specs=[
            pl.BlockSpec((_BN, f1), lambda i: (i, 0)),
            pl.BlockSpec((_BN, hp), lambda i: (i, 0)),
            pl.BlockSpec((hp, f1), lambda i: (0, 0)),
            pl.BlockSpec((1, f1), lambda i: (0, 0)),
            pl.BlockSpec((f1, f2 + hp), lambda i: (0, 0)),
            pl.BlockSpec((f1, hp), lambda i: (0, 0)),
        ],
        out_specs=[
            pl.BlockSpec((_BN, f2 + hp), lambda i: (i, 0)),
            pl.BlockSpec((_BN, hp), lambda i: (i, 0)),
        ],
        out_shape=[
            jax.ShapeDtypeStruct((n_pad, f2 + hp), jnp.float32),
            jax.ShapeDtypeStruct((n_pad, hp), jnp.float32),
        ],
    )(num1, den1, rep1, b1r, wsrc2, adst2)

    num2, den2 = _edge_scatter(zsrc2, zar2, src2d, dst2d, rep2, n_pad, e_pad, f2, hp)

    out = pl.pallas_call(
        _final_kernel,
        grid=(nt,),
        in_specs=[
            pl.BlockSpec((_BN, f2), lambda i: (i, 0)),
            pl.BlockSpec((_BN, hp), lambda i: (i, 0)),
            pl.BlockSpec((hp, f2), lambda i: (0, 0)),
            pl.BlockSpec((1, f2), lambda i: (0, 0)),
        ],
        out_specs=pl.BlockSpec((_BN, f2), lambda i: (i, 0)),
        out_shape=jax.ShapeDtypeStruct((n_pad, f2), jnp.float32),
    )(num2, den2, rep2, b2r)

    return out[:n]
